# Initial kernel scaffold; baseline (speedup 1.0000x reference)
import jax
import jax.numpy as jnp
from jax.experimental import pallas as pl


def kernel(x, router_logits, w13, w13_bias, w2, w2_bias):
    raise NotImplementedError("write your pallas kernel here")


# speedup vs baseline: 1.2241x; 1.2241x over previous
import jax
import jax.numpy as jnp
from jax.experimental import pallas as pl

_H = 128
_I = 128
_E = 8
_E_LOCAL = 2
_BT = 512


def _moe_block_kernel(x_ref, rl_ref, w13t_ref, w13b_ref, w2t_ref, w2b_ref, y_ref):
    x = x_ref[...]
    rl = rl_ref[...]
    neg_inf = jnp.float32(-jnp.inf)
    lane = jax.lax.broadcasted_iota(jnp.int32, rl.shape, 1)
    m1 = jnp.max(rl, axis=-1, keepdims=True)
    i1 = jnp.min(jnp.where(rl == m1, lane, _E), axis=-1, keepdims=True)
    masked = jnp.where(lane == i1, neg_inf, rl)
    m2 = jnp.max(masked, axis=-1, keepdims=True)
    i2 = jnp.min(jnp.where(masked == m2, lane, _E), axis=-1, keepdims=True)
    e2 = jnp.exp(m2 - m1)
    inv = 1.0 / (1.0 + e2)
    wa = inv
    wb = e2 * inv

    y = jnp.zeros((x.shape[0], _H), dtype=jnp.float32)
    for e in range(_E_LOCAL):
        comb = (wa * (i1 == e).astype(jnp.float32)
                + wb * (i2 == e).astype(jnp.float32))
        gu = (jnp.dot(x, w13t_ref[e], preferred_element_type=jnp.float32)
              + w13b_ref[e:e + 1, :])
        g = gu[:, :_I]
        u = gu[:, _I:]
        act = g * jax.nn.sigmoid(g) * u
        out = (jnp.dot(act, w2t_ref[e], preferred_element_type=jnp.float32)
               + w2b_ref[e:e + 1, :])
        y = y + comb * out
    y_ref[...] = y


def kernel(x, router_logits, w13, w13_bias, w2, w2_bias):
    T = x.shape[0]
    w13t = jnp.transpose(w13, (0, 2, 1))
    w2t = jnp.transpose(w2, (0, 2, 1))
    return pl.pallas_call(
        _moe_block_kernel,
        grid=(T // _BT,),
        in_specs=[
            pl.BlockSpec((_BT, _H), lambda i: (i, 0)),
            pl.BlockSpec((_BT, _E), lambda i: (i, 0)),
            pl.BlockSpec((_E_LOCAL, _H, 2 * _I), lambda i: (0, 0, 0)),
            pl.BlockSpec((_E_LOCAL, 2 * _I), lambda i: (0, 0)),
            pl.BlockSpec((_E_LOCAL, _I, _H), lambda i: (0, 0, 0)),
            pl.BlockSpec((_E_LOCAL, _H), lambda i: (0, 0)),
        ],
        out_specs=pl.BlockSpec((_BT, _H), lambda i: (i, 0)),
        out_shape=jax.ShapeDtypeStruct((T, _H), jnp.float32),
    )(x, router_logits, w13t, w13_bias, w2t, w2_bias)


# speedup vs baseline: 1.4179x; 1.1583x over previous
import jax
import jax.numpy as jnp
from jax.experimental import pallas as pl

_H = 128
_I = 128
_E = 8
_E_LOCAL = 2
_BT = 1024


def _moe_block_kernel(x_ref, rl_ref, w13c_ref, w13bc_ref, w2c_ref, w2b_ref, y_ref):
    x = x_ref[...]
    rl = rl_ref[...]
    neg_inf = jnp.float32(-jnp.inf)
    lane = jax.lax.broadcasted_iota(jnp.int32, rl.shape, 1)
    m1 = jnp.max(rl, axis=-1, keepdims=True)
    i1 = jnp.min(jnp.where(rl == m1, lane, _E), axis=-1, keepdims=True)
    masked = jnp.where(lane == i1, neg_inf, rl)
    m2 = jnp.max(masked, axis=-1, keepdims=True)
    i2 = jnp.min(jnp.where(masked == m2, lane, _E), axis=-1, keepdims=True)
    e2 = jnp.exp(m2 - m1)
    inv = 1.0 / (1.0 + e2)
    wa = inv
    wb = e2 * inv

    xb = x.astype(jnp.bfloat16)
    gu = (jnp.dot(xb, w13c_ref[...], preferred_element_type=jnp.float32)
          + w13bc_ref[...])
    acts = []
    ybias = jnp.zeros((x.shape[0], _H), dtype=jnp.float32)
    for e in range(_E_LOCAL):
        comb = (wa * (i1 == e).astype(jnp.float32)
                + wb * (i2 == e).astype(jnp.float32))
        g = gu[:, 2 * _I * e: 2 * _I * e + _I]
        u = gu[:, 2 * _I * e + _I: 2 * _I * (e + 1)]
        act = g * jax.nn.sigmoid(g) * u
        acts.append((comb * act).astype(jnp.bfloat16))
        ybias = ybias + comb * w2b_ref[e:e + 1, :]
    actcat = jnp.concatenate(acts, axis=-1)
    y_ref[...] = (jnp.dot(actcat, w2c_ref[...],
                          preferred_element_type=jnp.float32) + ybias)


def kernel(x, router_logits, w13, w13_bias, w2, w2_bias):
    T = x.shape[0]
    w13c = jnp.transpose(w13, (2, 0, 1)).reshape(_H, _E_LOCAL * 2 * _I)
    w13c = w13c.astype(jnp.bfloat16)
    w13bc = w13_bias.reshape(1, _E_LOCAL * 2 * _I)
    w2c = jnp.transpose(w2, (0, 2, 1)).reshape(_E_LOCAL * _I, _H)
    w2c = w2c.astype(jnp.bfloat16)
    return pl.pallas_call(
        _moe_block_kernel,
        grid=(T // _BT,),
        in_specs=[
            pl.BlockSpec((_BT, _H), lambda i: (i, 0)),
            pl.BlockSpec((_BT, _E), lambda i: (i, 0)),
            pl.BlockSpec((_H, _E_LOCAL * 2 * _I), lambda i: (0, 0)),
            pl.BlockSpec((1, _E_LOCAL * 2 * _I), lambda i: (0, 0)),
            pl.BlockSpec((_E_LOCAL * _I, _H), lambda i: (0, 0)),
            pl.BlockSpec((_E_LOCAL, _H), lambda i: (0, 0)),
        ],
        out_specs=pl.BlockSpec((_BT, _H), lambda i: (i, 0)),
        out_shape=jax.ShapeDtypeStruct((T, _H), jnp.float32),
    )(x, router_logits, w13c, w13bc, w2c, w2_bias)


# speedup vs baseline: 1.4885x; 1.0498x over previous
import jax
import jax.numpy as jnp
from jax.experimental import pallas as pl

_H = 128
_I = 128
_E = 8
_E_LOCAL = 2
_BT = 2048


def _moe_block_kernel(x_ref, rl_ref, w13c_ref, w13bc_ref, w2c_ref, w2b_ref, y_ref):
    x = x_ref[...]
    rl = rl_ref[...]
    neg_inf = jnp.float32(-jnp.inf)
    lane = jax.lax.broadcasted_iota(jnp.int32, rl.shape, 1)
    m1 = jnp.max(rl, axis=-1, keepdims=True)
    i1 = jnp.min(jnp.where(rl == m1, lane, _E), axis=-1, keepdims=True)
    masked = jnp.where(lane == i1, neg_inf, rl)
    m2 = jnp.max(masked, axis=-1, keepdims=True)
    i2 = jnp.min(jnp.where(masked == m2, lane, _E), axis=-1, keepdims=True)
    e2 = jnp.exp(m2 - m1)
    inv = 1.0 / (1.0 + e2)
    wa = inv
    wb = e2 * inv

    xb = x.astype(jnp.bfloat16)
    gu = (jnp.dot(xb, w13c_ref[...], preferred_element_type=jnp.float32)
          + w13bc_ref[...])
    acts = []
    ybias = jnp.zeros((x.shape[0], _H), dtype=jnp.float32)
    for e in range(_E_LOCAL):
        comb = (wa * (i1 == e).astype(jnp.float32)
                + wb * (i2 == e).astype(jnp.float32))
        g = gu[:, 2 * _I * e: 2 * _I * e + _I]
        u = gu[:, 2 * _I * e + _I: 2 * _I * (e + 1)]
        act = g * jax.nn.sigmoid(g) * u
        acts.append((comb * act).astype(jnp.bfloat16))
        ybias = ybias + comb * w2b_ref[e:e + 1, :]
    actcat = jnp.concatenate(acts, axis=-1)
    y_ref[...] = (jnp.dot(actcat, w2c_ref[...],
                          preferred_element_type=jnp.float32) + ybias)


def kernel(x, router_logits, w13, w13_bias, w2, w2_bias):
    T = x.shape[0]
    w13c = jnp.transpose(w13, (2, 0, 1)).reshape(_H, _E_LOCAL * 2 * _I)
    w13c = w13c.astype(jnp.bfloat16)
    w13bc = w13_bias.reshape(1, _E_LOCAL * 2 * _I)
    w2c = jnp.transpose(w2, (0, 2, 1)).reshape(_E_LOCAL * _I, _H)
    w2c = w2c.astype(jnp.bfloat16)
    return pl.pallas_call(
        _moe_block_kernel,
        grid=(T // _BT,),
        in_specs=[
            pl.BlockSpec((_BT, _H), lambda i: (i, 0)),
            pl.BlockSpec((_BT, _E), lambda i: (i, 0)),
            pl.BlockSpec((_H, _E_LOCAL * 2 * _I), lambda i: (0, 0)),
            pl.BlockSpec((1, _E_LOCAL * 2 * _I), lambda i: (0, 0)),
            pl.BlockSpec((_E_LOCAL * _I, _H), lambda i: (0, 0)),
            pl.BlockSpec((_E_LOCAL, _H), lambda i: (0, 0)),
        ],
        out_specs=pl.BlockSpec((_BT, _H), lambda i: (i, 0)),
        out_shape=jax.ShapeDtypeStruct((T, _H), jnp.float32),
    )(x, router_logits, w13c, w13bc, w2c, w2_bias)


# speedup vs baseline: 1.5174x; 1.0194x over previous
import jax
import jax.numpy as jnp
from jax.experimental import pallas as pl

_H = 128
_I = 128
_E = 8
_E_LOCAL = 2
_BT = 4096


def _moe_block_kernel(x_ref, rl_ref, w13c_ref, w13bc_ref, w2c_ref, w2b_ref, y_ref):
    x = x_ref[...]
    rl = rl_ref[...]
    neg_inf = jnp.float32(-jnp.inf)
    lane = jax.lax.broadcasted_iota(jnp.int32, rl.shape, 1)
    m1 = jnp.max(rl, axis=-1, keepdims=True)
    i1 = jnp.min(jnp.where(rl == m1, lane, _E), axis=-1, keepdims=True)
    masked = jnp.where(lane == i1, neg_inf, rl)
    m2 = jnp.max(masked, axis=-1, keepdims=True)
    i2 = jnp.min(jnp.where(masked == m2, lane, _E), axis=-1, keepdims=True)
    e2 = jnp.exp(m2 - m1)
    inv = 1.0 / (1.0 + e2)
    wa = inv
    wb = e2 * inv

    xb = x.astype(jnp.bfloat16)
    gu = (jnp.dot(xb, w13c_ref[...], preferred_element_type=jnp.float32)
          + w13bc_ref[...])
    acts = []
    ybias = jnp.zeros((x.shape[0], _H), dtype=jnp.float32)
    for e in range(_E_LOCAL):
        comb = (wa * (i1 == e).astype(jnp.float32)
                + wb * (i2 == e).astype(jnp.float32))
        g = gu[:, 2 * _I * e: 2 * _I * e + _I]
        u = gu[:, 2 * _I * e + _I: 2 * _I * (e + 1)]
        act = g * jax.nn.sigmoid(g) * u
        acts.append((comb * act).astype(jnp.bfloat16))
        ybias = ybias + comb * w2b_ref[e:e + 1, :]
    actcat = jnp.concatenate(acts, axis=-1)
    y_ref[...] = (jnp.dot(actcat, w2c_ref[...],
                          preferred_element_type=jnp.float32) + ybias)


def kernel(x, router_logits, w13, w13_bias, w2, w2_bias):
    T = x.shape[0]
    w13c = jnp.transpose(w13, (2, 0, 1)).reshape(_H, _E_LOCAL * 2 * _I)
    w13c = w13c.astype(jnp.bfloat16)
    w13bc = w13_bias.reshape(1, _E_LOCAL * 2 * _I)
    w2c = jnp.transpose(w2, (0, 2, 1)).reshape(_E_LOCAL * _I, _H)
    w2c = w2c.astype(jnp.bfloat16)
    return pl.pallas_call(
        _moe_block_kernel,
        grid=(T // _BT,),
        in_specs=[
            pl.BlockSpec((_BT, _H), lambda i: (i, 0)),
            pl.BlockSpec((_BT, _E), lambda i: (i, 0)),
            pl.BlockSpec((_H, _E_LOCAL * 2 * _I), lambda i: (0, 0)),
            pl.BlockSpec((1, _E_LOCAL * 2 * _I), lambda i: (0, 0)),
            pl.BlockSpec((_E_LOCAL * _I, _H), lambda i: (0, 0)),
            pl.BlockSpec((_E_LOCAL, _H), lambda i: (0, 0)),
        ],
        out_specs=pl.BlockSpec((_BT, _H), lambda i: (i, 0)),
        out_shape=jax.ShapeDtypeStruct((T, _H), jnp.float32),
    )(x, router_logits, w13c, w13bc, w2c, w2_bias)
